# baseline (speedup 1.0000x reference)
import functools

import jax
import jax.numpy as jnp
from jax import lax
from jax.experimental import pallas as pl
from jax.experimental.pallas import tpu as pltpu
from jax.experimental.pallas import tpu_sc as plsc

_B, _M, _N, _D = 16, 8192, 8192, 256
_L = 16
_NW = 32
_RPW = (_B * _M) // _NW
_CH = 64
_NCH = _RPW // _CH
_NB = 4


def _splat(v):
    return jnp.full((_L,), v)


def _sc_body(emb_hbm, num_hbm, pts_hbm, idx_hbm, ishape_hbm, w_hbm, out_hbm,
             num_v, idx_v, ishape_v, prow_v, w_v, g_v,
             b0, b1, b2, b3, si0, si1, si2, si3, so0, so1, so2, so3):
    bufs = (b0, b1, b2, b3)
    isems = (si0, si1, si2, si3)
    osems = (so0, so1, so2, so3)

    wid = lax.axis_index("s") * 2 + lax.axis_index("c")
    b = wid // 2
    rb0 = (wid % 2) * _RPW
    row0 = wid * _RPW

    pltpu.sync_copy(num_hbm, num_v)
    pltpu.sync_copy(idx_hbm, idx_v)
    pltpu.sync_copy(ishape_hbm, ishape_v)
    pltpu.sync_copy(w_hbm, w_v)

    lanes = lax.iota(jnp.int32, _L)
    bb = _splat(b)

    num_b = lax.reduce_max(plsc.load_gather(num_v, [bb]), axes=(0,))
    idx_b = lax.reduce_max(plsc.load_gather(idx_v, [bb]), axes=(0,))

    off5 = (b * _N + idx_b) * 5
    base = jnp.minimum((off5 // 8) * 8, _B * _N * 5 - 32)
    r = off5 - base
    pltpu.sync_copy(pts_hbm.at[pl.ds(base, 32)], prow_v)

    x = plsc.load_gather(prow_v, [_splat(r)])
    y = plsc.load_gather(prow_v, [_splat(r + 1)])
    ln = plsc.load_gather(prow_v, [_splat(r + 3)])
    an = plsc.load_gather(prow_v, [_splat(r + 4)])

    hf = plsc.load_gather(ishape_v, [_splat(2)]).astype(jnp.float32)
    wf = plsc.load_gather(ishape_v, [_splat(3)]).astype(jnp.float32)
    kp_scale = jnp.maximum(wf, hf) * 0.7
    v2 = wf * wf + hf * hf
    s = (v2 + 1.0) * 0.5
    for _ in range(30):
        s = 0.5 * (s + v2 / s)
    len_scale = s * 0.7

    nx = (x - wf * 0.5) / kp_scale
    ny = (y - hf * 0.5) / kp_scale
    na = (an - 45.0) / (180.0 * 0.7)
    nl = (ln - len_scale * 0.5) / len_scale

    for c in range(_D // _L):
        col = (c * _L + lanes) * 4
        w0 = plsc.load_gather(w_v, [col])
        w1 = plsc.load_gather(w_v, [col + 1])
        w2 = plsc.load_gather(w_v, [col + 2])
        w3 = plsc.load_gather(w_v, [col + 3])
        g_v[pl.ds(c * _L, _L)] = nx * w0 + ny * w1 + na * w2 + nl * w3

    gchunks = [g_v[pl.ds(c * _L, _L)] for c in range(_D // _L)]

    def start_in(ci, k):
        pltpu.async_copy(
            emb_hbm.at[pl.ds(row0 + ci * _CH, _CH), :], bufs[k], isems[k])

    def start_out(ci, k):
        pltpu.async_copy(
            bufs[k], out_hbm.at[pl.ds(row0 + ci * _CH, _CH), :], osems[k])

    def wait_in(k):
        pltpu.make_async_copy(emb_hbm.at[pl.ds(0, _CH), :], bufs[k],
                              isems[k]).wait()

    def wait_out(k):
        pltpu.make_async_copy(bufs[k], out_hbm.at[pl.ds(0, _CH), :],
                              osems[k]).wait()

    start_in(0, 0)
    start_in(1, 1)

    def add_rows(kadd, buf):
        def row_body(rr, carry):
            for c in range(_D // _L):
                sl = pl.ds(c * _L, _L)
                buf[rr, sl] = buf[rr, sl] + gchunks[c]
            return carry
        lax.fori_loop(0, kadd, row_body, 0)

    def group(t, carry):
        for st in range(_NB):
            ci = t * _NB + st
            wait_in(st)
            kadd = jnp.clip(num_b - (rb0 + ci * _CH), 0, _CH)
            add_rows(kadd, bufs[st])
            start_out(ci, st)
            nxt = ci + 2
            kn = (st + 2) % _NB

            @pl.when(nxt < _NCH)
            def _():
                @pl.when(nxt >= _NB)
                def _():
                    wait_out(kn)
                start_in(nxt, kn)
        return carry

    lax.fori_loop(0, _NCH // _NB, group, 0)

    for k in range(_NB):
        wait_out(k)


@jax.jit
def kernel(emb, num, pts, indeces, image_shape, W):
    num = num.astype(jnp.int32)
    indeces = indeces.astype(jnp.int32)
    image_shape = image_shape.astype(jnp.int32)

    emb2 = emb.reshape(_B * _M, _D)
    pts1 = pts.reshape(_B * _N * 5)
    w1 = W.reshape(_D * 4)

    mesh = plsc.VectorSubcoreMesh(core_axis_name="c", subcore_axis_name="s")
    run = functools.partial(
        pl.kernel,
        mesh=mesh,
        out_type=jax.ShapeDtypeStruct((_B * _M, _D), jnp.float32),
        scratch_types=[
            pltpu.VMEM((_B,), jnp.int32),
            pltpu.VMEM((_B,), jnp.int32),
            pltpu.VMEM((4,), jnp.int32),
            pltpu.VMEM((32,), jnp.float32),
            pltpu.VMEM((_D * 4,), jnp.float32),
            pltpu.VMEM((_D,), jnp.float32),
        ]
        + [pltpu.VMEM((_CH, _D), jnp.float32)] * _NB
        + [pltpu.SemaphoreType.DMA] * (2 * _NB),
        compiler_params=pltpu.CompilerParams(needs_layout_passes=False),
    )(_sc_body)

    out2 = run(emb2, num, pts1, indeces, image_shape, w1)
    return out2.reshape(_B, _M, _D)


# speedup vs baseline: 1.0622x; 1.0622x over previous
import functools

import jax
import jax.numpy as jnp
from jax import lax
from jax.experimental import pallas as pl
from jax.experimental.pallas import tpu as pltpu
from jax.experimental.pallas import tpu_sc as plsc

_B, _M, _N, _D = 16, 8192, 8192, 256
_L = 16
_BM = 8192


def _splat(v):
    return jnp.full((_L,), v)


def _sc_proj_body(pts_hbm, idx_hbm, ishape_hbm, w_hbm, g_hbm,
                  idx_v, ishape_v, prow_v, w_v, g_v):
    wid = lax.axis_index("s") * 2 + lax.axis_index("c")

    @pl.when(wid < _B)
    def _():
        b = wid
        pltpu.sync_copy(idx_hbm, idx_v)
        pltpu.sync_copy(ishape_hbm, ishape_v)
        pltpu.sync_copy(w_hbm, w_v)

        lanes = lax.iota(jnp.int32, _L)
        idx_b = lax.reduce_max(plsc.load_gather(idx_v, [_splat(b)]), axes=(0,))

        off5 = (b * _N + idx_b) * 5
        base = jnp.minimum((off5 // 8) * 8, _B * _N * 5 - 32)
        r = off5 - base
        pltpu.sync_copy(pts_hbm.at[pl.ds(base, 32)], prow_v)

        x = plsc.load_gather(prow_v, [_splat(r)])
        y = plsc.load_gather(prow_v, [_splat(r + 1)])
        ln = plsc.load_gather(prow_v, [_splat(r + 3)])
        an = plsc.load_gather(prow_v, [_splat(r + 4)])

        hf = plsc.load_gather(ishape_v, [_splat(2)]).astype(jnp.float32)
        wf = plsc.load_gather(ishape_v, [_splat(3)]).astype(jnp.float32)
        kp_scale = jnp.maximum(wf, hf) * 0.7
        v2 = wf * wf + hf * hf
        s = (v2 + 1.0) * 0.5
        for _ in range(30):
            s = 0.5 * (s + v2 / s)
        len_scale = s * 0.7

        nx = (x - wf * 0.5) / kp_scale
        ny = (y - hf * 0.5) / kp_scale
        na = (an - 45.0) / (180.0 * 0.7)
        nl = (ln - len_scale * 0.5) / len_scale

        for c in range(_D // _L):
            col = (c * _L + lanes) * 4
            w0 = plsc.load_gather(w_v, [col])
            w1 = plsc.load_gather(w_v, [col + 1])
            w2 = plsc.load_gather(w_v, [col + 2])
            w3 = plsc.load_gather(w_v, [col + 3])
            g_v[pl.ds(c * _L, _L)] = nx * w0 + ny * w1 + na * w2 + nl * w3

        pltpu.sync_copy(g_v, g_hbm.at[b, 0, :])


def _tc_stream_body(num_ref, g_ref, emb_ref, out_ref):
    b = pl.program_id(0)
    row = lax.broadcasted_iota(jnp.int32, (_BM, 1), 0)
    mask = row < num_ref[b]
    out_ref[0] = emb_ref[0] + jnp.where(mask, g_ref[0], 0.0)


@jax.jit
def kernel(emb, num, pts, indeces, image_shape, W):
    num = num.astype(jnp.int32)
    indeces = indeces.astype(jnp.int32)
    image_shape = image_shape.astype(jnp.int32)

    pts1 = pts.reshape(_B * _N * 5)
    w1 = W.reshape(_D * 4)

    mesh = plsc.VectorSubcoreMesh(core_axis_name="c", subcore_axis_name="s")
    g = pl.kernel(
        _sc_proj_body,
        mesh=mesh,
        out_type=jax.ShapeDtypeStruct((_B, 1, _D), jnp.float32),
        scratch_types=[
            pltpu.VMEM((_B,), jnp.int32),
            pltpu.VMEM((4,), jnp.int32),
            pltpu.VMEM((32,), jnp.float32),
            pltpu.VMEM((_D * 4,), jnp.float32),
            pltpu.VMEM((_D,), jnp.float32),
        ],
        compiler_params=pltpu.CompilerParams(needs_layout_passes=False),
    )(pts1, indeces, image_shape, w1)

    return pl.pallas_call(
        _tc_stream_body,
        grid_spec=pltpu.PrefetchScalarGridSpec(
            num_scalar_prefetch=1,
            grid=(_B,),
            in_specs=[
                pl.BlockSpec((1, 1, _D), lambda b, n: (b, 0, 0)),
                pl.BlockSpec((1, _BM, _D), lambda b, n: (b, 0, 0)),
            ],
            out_specs=pl.BlockSpec((1, _BM, _D), lambda b, n: (b, 0, 0)),
        ),
        out_shape=jax.ShapeDtypeStruct((_B, _M, _D), emb.dtype),
        compiler_params=pltpu.CompilerParams(
            dimension_semantics=("parallel",),
        ),
    )(num, g, emb)


# speedup vs baseline: 1.7162x; 1.6157x over previous
import jax
import jax.numpy as jnp
from jax.experimental import pallas as pl
from jax.experimental.pallas import tpu as pltpu

_B, _M, _N, _D = 16, 8192, 8192, 256
_BM = 8192
_PR = 8


def _body(idx_ref, num_ref, ishape_ref, pts_ref, wt_ref, emb_ref, out_ref):
    b = pl.program_id(0)
    j = pl.program_id(1)

    hf = ishape_ref[2].astype(jnp.float32)
    wf = ishape_ref[3].astype(jnp.float32)
    kp_scale = jnp.maximum(wf, hf) * 0.7
    max_len = jnp.sqrt(wf * wf + hf * hf)
    len_scale = max_len * 0.7

    r = idx_ref[b] % _PR
    x = pts_ref[0, r, 0]
    y = pts_ref[0, r, 1]
    ln = pts_ref[0, r, 3]
    an = pts_ref[0, r, 4]

    nx = (x - wf * 0.5) / kp_scale
    ny = (y - hf * 0.5) / kp_scale
    na = (an - 45.0) / (180.0 * 0.7)
    nl = (ln - len_scale * 0.5) / len_scale

    g = (nx * wt_ref[0:1, :] + ny * wt_ref[1:2, :]
         + na * wt_ref[2:3, :] + nl * wt_ref[3:4, :])

    row = jax.lax.broadcasted_iota(jnp.int32, (_BM, 1), 0) + j * _BM
    mask = row < num_ref[b]
    out_ref[0] = emb_ref[0] + jnp.where(mask, g, 0.0)


@jax.jit
def kernel(emb, num, pts, indeces, image_shape, W):
    num = num.astype(jnp.int32)
    indeces = indeces.astype(jnp.int32)
    image_shape = image_shape.astype(jnp.int32)
    wt = W.T

    grid = (_B, _M // _BM)
    return pl.pallas_call(
        _body,
        grid_spec=pltpu.PrefetchScalarGridSpec(
            num_scalar_prefetch=3,
            grid=grid,
            in_specs=[
                pl.BlockSpec((1, _PR, 5),
                             lambda b, j, idx, n, s: (b, idx[b] // _PR, 0)),
                pl.BlockSpec((4, _D), lambda b, j, idx, n, s: (0, 0)),
                pl.BlockSpec((1, _BM, _D), lambda b, j, idx, n, s: (b, j, 0)),
            ],
            out_specs=pl.BlockSpec((1, _BM, _D),
                                   lambda b, j, idx, n, s: (b, j, 0)),
        ),
        out_shape=jax.ShapeDtypeStruct((_B, _M, _D), emb.dtype),
        compiler_params=pltpu.CompilerParams(
            dimension_semantics=("parallel", "parallel"),
        ),
    )(indeces, num, image_shape, pts, wt, emb)
